# Initial kernel scaffold; baseline (speedup 1.0000x reference)
import jax
import jax.numpy as jnp
from jax.experimental import pallas as pl


def kernel(x, weight, bias):
    raise NotImplementedError("write your pallas kernel here")


# speedup vs baseline: 1.0015x; 1.0015x over previous
import jax
import jax.numpy as jnp
from jax import lax
from jax.experimental import pallas as pl
from jax.experimental.pallas import tpu as pltpu

_LANE = 128


def _round_up(n, m):
    return ((n + m - 1) // m) * m


def _affine_kernel(b_ref, x_ref, w_ref, o_ref):
    acc = jnp.dot(x_ref[...].astype(jnp.bfloat16), w_ref[...],
                  preferred_element_type=jnp.float32)
    o_ref[...] = acc + b_ref[0]


def kernel(x, weight, bias):
    B = x.shape[0]
    xf = x.astype(jnp.float32)
    b_pad = _round_up(B, _LANE)
    if b_pad != B:
        xf = jnp.pad(xf, ((0, b_pad - B), (0, 0)))
    rows = b_pad // _LANE
    xi = xf.reshape(rows, 2 * _LANE)

    wmat = jnp.kron(jnp.eye(_LANE, dtype=jnp.float32),
                    weight.astype(jnp.float32).reshape(2, 1))
    wmat = wmat.astype(jnp.bfloat16)

    tile = 2048
    while tile > 8 and rows % tile != 0:
        tile //= 2
    if rows % tile != 0:
        tile = rows
    num_tiles = rows // tile

    out = pl.pallas_call(
        _affine_kernel,
        out_shape=jax.ShapeDtypeStruct((rows, _LANE), jnp.float32),
        grid=(num_tiles,),
        in_specs=[
            pl.BlockSpec(memory_space=pltpu.MemorySpace.SMEM),
            pl.BlockSpec((tile, 2 * _LANE), lambda i: (i, 0)),
            pl.BlockSpec((2 * _LANE, _LANE), lambda i: (0, 0)),
        ],
        out_specs=pl.BlockSpec((tile, _LANE), lambda i: (i, 0)),
        compiler_params=pltpu.CompilerParams(
            dimension_semantics=("parallel",),
        ),
    )(bias.astype(jnp.float32).reshape(-1), xi, wmat)

    return out.reshape(b_pad)[:B].reshape(B, 1)


# speedup vs baseline: 1.2998x; 1.2978x over previous
import jax
import jax.numpy as jnp
from jax.experimental import pallas as pl
from jax.experimental.pallas import tpu as pltpu


def _affine_kernel(w_ref, b_ref, x_ref, o_ref):
    w0 = w_ref[0, 0]
    w1 = w_ref[0, 1]
    b = b_ref[0]
    x0 = x_ref[:, 0:1]
    x1 = x_ref[:, 1:2]
    o_ref[...] = x0 * w0 + x1 * w1 + b


def kernel(x, weight, bias):
    B = x.shape[0]
    xf = x.astype(jnp.float32)

    tile = 2048
    while tile > 8 and B % tile != 0:
        tile //= 2
    if B % tile != 0:
        tile = B
    num_tiles = B // tile

    return pl.pallas_call(
        _affine_kernel,
        out_shape=jax.ShapeDtypeStruct((B, 1), jnp.float32),
        grid=(num_tiles,),
        in_specs=[
            pl.BlockSpec(memory_space=pltpu.MemorySpace.SMEM),
            pl.BlockSpec(memory_space=pltpu.MemorySpace.SMEM),
            pl.BlockSpec((tile, 2), lambda i: (i, 0)),
        ],
        out_specs=pl.BlockSpec((tile, 1), lambda i: (i, 0)),
        compiler_params=pltpu.CompilerParams(
            dimension_semantics=("parallel",),
        ),
    )(weight.astype(jnp.float32), bias.astype(jnp.float32), xf)


# speedup vs baseline: 1.6581x; 1.2757x over previous
import jax
import jax.numpy as jnp
from jax.experimental import pallas as pl
from jax.experimental.pallas import tpu as pltpu


def _affine_kernel(w_ref, b_ref, x_ref, o_ref):
    w0 = w_ref[0, 0]
    w1 = w_ref[0, 1]
    b = b_ref[0]
    x0 = x_ref[:, 0:1]
    x1 = x_ref[:, 1:2]
    o_ref[...] = x0 * w0 + x1 * w1 + b


def kernel(x, weight, bias):
    B = x.shape[0]
    xf = x.astype(jnp.float32)

    tile = 8192
    while tile > 8 and B % tile != 0:
        tile //= 2
    if B % tile != 0:
        tile = B
    num_tiles = B // tile

    return pl.pallas_call(
        _affine_kernel,
        out_shape=jax.ShapeDtypeStruct((B, 1), jnp.float32),
        grid=(num_tiles,),
        in_specs=[
            pl.BlockSpec(memory_space=pltpu.MemorySpace.SMEM),
            pl.BlockSpec(memory_space=pltpu.MemorySpace.SMEM),
            pl.BlockSpec((tile, 2), lambda i: (i, 0)),
        ],
        out_specs=pl.BlockSpec((tile, 1), lambda i: (i, 0)),
        compiler_params=pltpu.CompilerParams(
            dimension_semantics=("parallel",),
        ),
    )(weight.astype(jnp.float32), bias.astype(jnp.float32), xf)


# speedup vs baseline: 1.6740x; 1.0096x over previous
import jax
import jax.numpy as jnp
from jax.experimental import pallas as pl
from jax.experimental.pallas import tpu as pltpu


def _affine_kernel(w_ref, b_ref, x_ref, o_ref):
    w0 = w_ref[0, 0]
    w1 = w_ref[0, 1]
    b = b_ref[0]
    x0 = x_ref[:, 0:1]
    x1 = x_ref[:, 1:2]
    o_ref[...] = x0 * w0 + x1 * w1 + b


def kernel(x, weight, bias):
    B = x.shape[0]
    xf = x.astype(jnp.float32)

    tile = 16384
    while tile > 8 and B % tile != 0:
        tile //= 2
    if B % tile != 0:
        tile = B
    num_tiles = B // tile

    return pl.pallas_call(
        _affine_kernel,
        out_shape=jax.ShapeDtypeStruct((B, 1), jnp.float32),
        grid=(num_tiles,),
        in_specs=[
            pl.BlockSpec(memory_space=pltpu.MemorySpace.SMEM),
            pl.BlockSpec(memory_space=pltpu.MemorySpace.SMEM),
            pl.BlockSpec((tile, 2), lambda i: (i, 0)),
        ],
        out_specs=pl.BlockSpec((tile, 1), lambda i: (i, 0)),
        compiler_params=pltpu.CompilerParams(
            dimension_semantics=("parallel",),
        ),
    )(weight.astype(jnp.float32), bias.astype(jnp.float32), xf)


# speedup vs baseline: 50.0464x; 29.8965x over previous
import jax
import jax.numpy as jnp
from jax.experimental import pallas as pl
from jax.experimental.pallas import tpu as pltpu


def _affine_dense_kernel(w_ref, b_ref, x_ref, o_ref):
    x0 = x_ref[0, 0]
    x1 = x_ref[1, 0]
    o_ref[0] = x0 * w_ref[0, 0] + x1 * w_ref[0, 1] + b_ref[0]


def _affine_narrow_kernel(w_ref, b_ref, x_ref, o_ref):
    x0 = x_ref[:, 0:1]
    x1 = x_ref[:, 1:2]
    o_ref[...] = x0 * w_ref[0, 0] + x1 * w_ref[0, 1] + b_ref[0]


def _narrow_path(xf, weight, bias):
    B = xf.shape[0]
    tile = 16384
    while tile > 8 and B % tile != 0:
        tile //= 2
    if B % tile != 0:
        tile = B
    return pl.pallas_call(
        _affine_narrow_kernel,
        out_shape=jax.ShapeDtypeStruct((B, 1), jnp.float32),
        grid=(B // tile,),
        in_specs=[
            pl.BlockSpec(memory_space=pltpu.MemorySpace.SMEM),
            pl.BlockSpec(memory_space=pltpu.MemorySpace.SMEM),
            pl.BlockSpec((tile, 2), lambda i: (i, 0)),
        ],
        out_specs=pl.BlockSpec((tile, 1), lambda i: (i, 0)),
        compiler_params=pltpu.CompilerParams(
            dimension_semantics=("parallel",),
        ),
    )(weight, bias, xf)


def kernel(x, weight, bias):
    B = x.shape[0]
    xf = x.astype(jnp.float32)
    wf = weight.astype(jnp.float32)
    bf = bias.astype(jnp.float32)

    lo = 16384
    while lo > 128 and B % (8 * lo) != 0:
        lo //= 2
    if B % (8 * lo) != 0:
        return _narrow_path(xf, wf, bf)
    nb = B // (8 * lo)

    xv = xf.T.reshape(2, nb, 8, lo)
    out = pl.pallas_call(
        _affine_dense_kernel,
        out_shape=jax.ShapeDtypeStruct((nb, 8, lo), jnp.float32),
        grid=(nb,),
        in_specs=[
            pl.BlockSpec(memory_space=pltpu.MemorySpace.SMEM),
            pl.BlockSpec(memory_space=pltpu.MemorySpace.SMEM),
            pl.BlockSpec((2, 1, 8, lo), lambda i: (0, i, 0, 0)),
        ],
        out_specs=pl.BlockSpec((1, 8, lo), lambda i: (i, 0, 0)),
        compiler_params=pltpu.CompilerParams(
            dimension_semantics=("parallel",),
        ),
    )(wf, bf, xv)
    return out.reshape(B, 1)


# speedup vs baseline: 54.0461x; 1.0799x over previous
import jax
import jax.numpy as jnp
from jax.experimental import pallas as pl
from jax.experimental.pallas import tpu as pltpu


def _affine_dense_kernel(w_ref, b_ref, x_ref, o_ref):
    x0 = x_ref[0, 0]
    x1 = x_ref[1, 0]
    o_ref[0] = x0 * w_ref[0, 0] + x1 * w_ref[0, 1] + b_ref[0]


def _affine_narrow_kernel(w_ref, b_ref, x_ref, o_ref):
    x0 = x_ref[:, 0:1]
    x1 = x_ref[:, 1:2]
    o_ref[...] = x0 * w_ref[0, 0] + x1 * w_ref[0, 1] + b_ref[0]


def _narrow_path(xf, weight, bias):
    B = xf.shape[0]
    tile = 16384
    while tile > 8 and B % tile != 0:
        tile //= 2
    if B % tile != 0:
        tile = B
    return pl.pallas_call(
        _affine_narrow_kernel,
        out_shape=jax.ShapeDtypeStruct((B, 1), jnp.float32),
        grid=(B // tile,),
        in_specs=[
            pl.BlockSpec(memory_space=pltpu.MemorySpace.SMEM),
            pl.BlockSpec(memory_space=pltpu.MemorySpace.SMEM),
            pl.BlockSpec((tile, 2), lambda i: (i, 0)),
        ],
        out_specs=pl.BlockSpec((tile, 1), lambda i: (i, 0)),
        compiler_params=pltpu.CompilerParams(
            dimension_semantics=("parallel",),
        ),
    )(weight, bias, xf)


def kernel(x, weight, bias):
    B = x.shape[0]
    xf = x.astype(jnp.float32)
    wf = weight.astype(jnp.float32)
    bf = bias.astype(jnp.float32)

    lo = 32768
    while lo > 128 and B % (8 * lo) != 0:
        lo //= 2
    if B % (8 * lo) != 0:
        return _narrow_path(xf, wf, bf)
    nb = B // (8 * lo)

    xv = xf.T.reshape(2, nb, 8, lo)
    out = pl.pallas_call(
        _affine_dense_kernel,
        out_shape=jax.ShapeDtypeStruct((nb, 8, lo), jnp.float32),
        grid=(nb,),
        in_specs=[
            pl.BlockSpec(memory_space=pltpu.MemorySpace.SMEM),
            pl.BlockSpec(memory_space=pltpu.MemorySpace.SMEM),
            pl.BlockSpec((2, 1, 8, lo), lambda i: (0, i, 0, 0)),
        ],
        out_specs=pl.BlockSpec((1, 8, lo), lambda i: (i, 0, 0)),
        compiler_params=pltpu.CompilerParams(
            dimension_semantics=("parallel",),
        ),
    )(wf, bf, xv)
    return out.reshape(B, 1)


# speedup vs baseline: 55.9499x; 1.0352x over previous
import jax
import jax.numpy as jnp
from jax.experimental import pallas as pl
from jax.experimental.pallas import tpu as pltpu


def _affine_dense_kernel(w_ref, b_ref, x_ref, o_ref):
    x0 = x_ref[0, 0]
    x1 = x_ref[1, 0]
    o_ref[0] = x0 * w_ref[0, 0] + x1 * w_ref[0, 1] + b_ref[0]


def _affine_narrow_kernel(w_ref, b_ref, x_ref, o_ref):
    x0 = x_ref[:, 0:1]
    x1 = x_ref[:, 1:2]
    o_ref[...] = x0 * w_ref[0, 0] + x1 * w_ref[0, 1] + b_ref[0]


def _narrow_path(xf, weight, bias):
    B = xf.shape[0]
    tile = 16384
    while tile > 8 and B % tile != 0:
        tile //= 2
    if B % tile != 0:
        tile = B
    return pl.pallas_call(
        _affine_narrow_kernel,
        out_shape=jax.ShapeDtypeStruct((B, 1), jnp.float32),
        grid=(B // tile,),
        in_specs=[
            pl.BlockSpec(memory_space=pltpu.MemorySpace.SMEM),
            pl.BlockSpec(memory_space=pltpu.MemorySpace.SMEM),
            pl.BlockSpec((tile, 2), lambda i: (i, 0)),
        ],
        out_specs=pl.BlockSpec((tile, 1), lambda i: (i, 0)),
        compiler_params=pltpu.CompilerParams(
            dimension_semantics=("parallel",),
        ),
    )(weight, bias, xf)


def kernel(x, weight, bias):
    B = x.shape[0]
    xf = x.astype(jnp.float32)
    wf = weight.astype(jnp.float32)
    bf = bias.astype(jnp.float32)

    lo = 65536
    while lo > 128 and B % (8 * lo) != 0:
        lo //= 2
    if B % (8 * lo) != 0:
        return _narrow_path(xf, wf, bf)
    nb = B // (8 * lo)

    xv = xf.T.reshape(2, nb, 8, lo)
    out = pl.pallas_call(
        _affine_dense_kernel,
        out_shape=jax.ShapeDtypeStruct((nb, 8, lo), jnp.float32),
        grid=(nb,),
        in_specs=[
            pl.BlockSpec(memory_space=pltpu.MemorySpace.SMEM),
            pl.BlockSpec(memory_space=pltpu.MemorySpace.SMEM),
            pl.BlockSpec((2, 1, 8, lo), lambda i: (0, i, 0, 0)),
        ],
        out_specs=pl.BlockSpec((1, 8, lo), lambda i: (i, 0, 0)),
        compiler_params=pltpu.CompilerParams(
            dimension_semantics=("parallel",),
        ),
    )(wf, bf, xv)
    return out.reshape(B, 1)
